# baseline (speedup 1.0000x reference)
import functools

import jax
import jax.numpy as jnp
from jax import lax
from jax.experimental import pallas as pl
from jax.experimental.pallas import tpu as pltpu
from jax.experimental.pallas import tpu_sc as plsc

_VOCAB = 1000000
_D = 64
_C = 2
_B = 4096
_SEQ = 200

_NC = 2
_NS = 16
_NW = _NC * _NS
_BPW = _B // _NW
_LANES = 16
_GPW = _BPW // _LANES

_NB = 32768
_GRID = (_VOCAB + _NB - 1) // _NB


def _proj_body(w_ref, b_ref, tt_ref, o_ref):
    scale = jnp.float32(1.0 / _SEQ)
    res = (jnp.dot(w_ref[...], tt_ref[...],
                   preferred_element_type=jnp.float32)
           + b_ref[...]) * scale
    au = lax.bitcast_convert_type(
        res[0, :].astype(jnp.bfloat16), jnp.uint16).astype(jnp.uint32)
    bu = lax.bitcast_convert_type(
        res[1, :].astype(jnp.bfloat16), jnp.uint16).astype(jnp.uint32)
    o_ref[...] = lax.bitcast_convert_type((au << 16) | bu, jnp.float32)


_proj = pl.pallas_call(
    _proj_body,
    grid=(_GRID,),
    in_specs=[
        pl.BlockSpec((_C, _D), lambda i: (0, 0)),
        pl.BlockSpec((_C, 1), lambda i: (0, 0)),
        pl.BlockSpec((_D, _NB), lambda i: (0, i)),
    ],
    out_specs=pl.BlockSpec((_NB,), lambda i: (i,)),
    out_shape=jax.ShapeDtypeStruct((_VOCAB,), jnp.float32),
)

_WAVE = 10
_NWAVE = _SEQ // _WAVE
_HIMASK = jnp.uint32(0xFFFF0000)


def _pool_body(xt_hbm, tp_hbm, out_hbm,
               idx_v, g_v, o0_v, o1_v, sem0, sem1):
    wid = lax.axis_index("s") * _NC + lax.axis_index("c")
    base = wid * _BPW

    pltpu.sync_copy(xt_hbm.at[:, pl.ds(base, _BPW)], idx_v)

    sems = (sem0, sem1)

    def copy(j, sem):
        return pltpu.make_async_copy(tp_hbm.at[idx_v.at[j]],
                                     g_v.at[pl.ds(j * _BPW, _BPW)], sem)

    def fire_wave(w, par):
        def f1(j, _):
            copy(j, sems[par]).start()
            return 0
        lax.fori_loop(w * _WAVE, (w + 1) * _WAVE, f1, 0)

    def wait_wave(w, par):
        def f1(j, _):
            copy(j, sems[par]).wait()
            return 0
        lax.fori_loop(w * _WAVE, (w + 1) * _WAVE, f1, 0)

    def reduce_wave(w, accs):
        def rbody(j, carry):
            acc = list(carry)
            for g in range(_GPW):
                u = plsc.bitcast(g_v[pl.ds(j * _BPW + g * _LANES, _LANES)],
                                 jnp.uint32)
                c0 = plsc.bitcast(u & _HIMASK, jnp.float32)
                c1 = plsc.bitcast(u << 16, jnp.float32)
                acc[g] = acc[g] + c0
                acc[_GPW + g] = acc[_GPW + g] + c1
            return tuple(acc)
        return lax.fori_loop(w * _WAVE, (w + 1) * _WAVE, rbody, accs)

    fire_wave(0, 0)
    zero = jnp.zeros((_LANES,), jnp.float32)

    def outer(i, carry):
        for par in range(2):
            w = 2 * i + par

            @pl.when(w + 1 < _NWAVE)
            def _fire():
                fire_wave(w + 1, 1 - par)

            wait_wave(w, par)
            carry = reduce_wave(w, carry)
        return carry

    acc = lax.fori_loop(0, _NWAVE // 2, outer, (zero,) * (2 * _GPW))
    for g in range(_GPW):
        o0_v[pl.ds(g * _LANES, _LANES)] = acc[g]
        o1_v[pl.ds(g * _LANES, _LANES)] = acc[_GPW + g]

    pltpu.sync_copy(o0_v, out_hbm.at[pl.ds(base, _BPW)])
    pltpu.sync_copy(o1_v, out_hbm.at[pl.ds(_B + base, _BPW)])


@functools.partial(
    pl.kernel,
    out_type=jax.ShapeDtypeStruct((_C * _B,), jnp.float32),
    mesh=plsc.VectorSubcoreMesh(core_axis_name="c", subcore_axis_name="s"),
    scratch_types=[
        pltpu.VMEM((_SEQ, _BPW), jnp.int32),
        pltpu.VMEM((_SEQ * _BPW,), jnp.float32),
        pltpu.VMEM((_BPW,), jnp.float32),
        pltpu.VMEM((_BPW,), jnp.float32),
        pltpu.SemaphoreType.DMA,
        pltpu.SemaphoreType.DMA,
    ],
    compiler_params=pltpu.CompilerParams(use_tc_tiling_on_sc=False,
                                         needs_layout_passes=False),
)
def _sc_pool(xt_hbm, tp_hbm, out_hbm,
             idx_v, g_v, o0_v, o1_v, sem0, sem1):
    _pool_body(xt_hbm, tp_hbm, out_hbm,
               idx_v, g_v, o0_v, o1_v, sem0, sem1)


def kernel(x, table, W, b):
    tp = _proj(W.astype(jnp.float32), b.astype(jnp.float32)[:, None],
               table.T)
    out = _sc_pool(x.T.astype(jnp.int32), tp)
    return out.reshape(_C, _B).T


# speedup vs baseline: 1.0056x; 1.0056x over previous
import functools

import jax
import jax.numpy as jnp
from jax import lax
from jax.experimental import pallas as pl
from jax.experimental.pallas import tpu as pltpu
from jax.experimental.pallas import tpu_sc as plsc

_VOCAB = 1000000
_D = 64
_C = 2
_B = 4096
_SEQ = 200

_NC = 2
_NS = 16
_NW = _NC * _NS
_BPW = _B // _NW
_LANES = 16
_GPW = _BPW // _LANES

_NB = 32768
_GRID = (_VOCAB + _NB - 1) // _NB


def _proj_body(w_ref, b_ref, tt_ref, o_ref):
    scale = jnp.float32(1.0 / _SEQ)
    res = (jnp.dot(w_ref[...], tt_ref[...],
                   preferred_element_type=jnp.float32)
           + b_ref[...]) * scale
    au = lax.bitcast_convert_type(
        res[0, :].astype(jnp.bfloat16), jnp.uint16).astype(jnp.uint32)
    bu = lax.bitcast_convert_type(
        res[1, :].astype(jnp.bfloat16), jnp.uint16).astype(jnp.uint32)
    o_ref[...] = lax.bitcast_convert_type((au << 16) | bu, jnp.float32)


_proj = pl.pallas_call(
    _proj_body,
    grid=(_GRID,),
    in_specs=[
        pl.BlockSpec((_C, _D), lambda i: (0, 0)),
        pl.BlockSpec((_C, 1), lambda i: (0, 0)),
        pl.BlockSpec((_D, _NB), lambda i: (0, i)),
    ],
    out_specs=pl.BlockSpec((_NB,), lambda i: (i,)),
    out_shape=jax.ShapeDtypeStruct((_VOCAB,), jnp.float32),
)

_WAVE = 25
_NWAVE = _SEQ // _WAVE
_HIMASK = jnp.uint32(0xFFFF0000)


def _pool_body(xt_hbm, tp_hbm, out_hbm,
               idx_v, g_v, o0_v, o1_v, sem0, sem1):
    wid = lax.axis_index("s") * _NC + lax.axis_index("c")
    base = wid * _BPW

    pltpu.sync_copy(xt_hbm.at[:, pl.ds(base, _BPW)], idx_v)

    sems = (sem0, sem1)

    def copy(j, sem):
        return pltpu.make_async_copy(tp_hbm.at[idx_v.at[j]],
                                     g_v.at[pl.ds(j * _BPW, _BPW)], sem)

    def fire_wave(w, par):
        def f1(j, _):
            copy(j, sems[par]).start()
            return 0
        lax.fori_loop(w * _WAVE, (w + 1) * _WAVE, f1, 0)

    def wait_wave(w, par):
        def f1(j, _):
            copy(j, sems[par]).wait()
            return 0
        lax.fori_loop(w * _WAVE, (w + 1) * _WAVE, f1, 0)

    def reduce_wave(w, accs):
        def rbody(j, carry):
            acc = list(carry)
            for g in range(_GPW):
                u = plsc.bitcast(g_v[pl.ds(j * _BPW + g * _LANES, _LANES)],
                                 jnp.uint32)
                c0 = plsc.bitcast(u & _HIMASK, jnp.float32)
                c1 = plsc.bitcast(u << 16, jnp.float32)
                acc[g] = acc[g] + c0
                acc[_GPW + g] = acc[_GPW + g] + c1
            return tuple(acc)
        return lax.fori_loop(w * _WAVE, (w + 1) * _WAVE, rbody, accs)

    fire_wave(0, 0)
    zero = jnp.zeros((_LANES,), jnp.float32)

    def outer(i, carry):
        for par in range(2):
            w = 2 * i + par

            @pl.when(w + 1 < _NWAVE)
            def _fire():
                fire_wave(w + 1, 1 - par)

            wait_wave(w, par)
            carry = reduce_wave(w, carry)
        return carry

    acc = lax.fori_loop(0, _NWAVE // 2, outer, (zero,) * (2 * _GPW))
    for g in range(_GPW):
        o0_v[pl.ds(g * _LANES, _LANES)] = acc[g]
        o1_v[pl.ds(g * _LANES, _LANES)] = acc[_GPW + g]

    pltpu.sync_copy(o0_v, out_hbm.at[pl.ds(base, _BPW)])
    pltpu.sync_copy(o1_v, out_hbm.at[pl.ds(_B + base, _BPW)])


@functools.partial(
    pl.kernel,
    out_type=jax.ShapeDtypeStruct((_C * _B,), jnp.float32),
    mesh=plsc.VectorSubcoreMesh(core_axis_name="c", subcore_axis_name="s"),
    scratch_types=[
        pltpu.VMEM((_SEQ, _BPW), jnp.int32),
        pltpu.VMEM((_SEQ * _BPW,), jnp.float32),
        pltpu.VMEM((_BPW,), jnp.float32),
        pltpu.VMEM((_BPW,), jnp.float32),
        pltpu.SemaphoreType.DMA,
        pltpu.SemaphoreType.DMA,
    ],
    compiler_params=pltpu.CompilerParams(use_tc_tiling_on_sc=False,
                                         needs_layout_passes=False),
)
def _sc_pool(xt_hbm, tp_hbm, out_hbm,
             idx_v, g_v, o0_v, o1_v, sem0, sem1):
    _pool_body(xt_hbm, tp_hbm, out_hbm,
               idx_v, g_v, o0_v, o1_v, sem0, sem1)


def kernel(x, table, W, b):
    tp = _proj(W.astype(jnp.float32), b.astype(jnp.float32)[:, None],
               table.T)
    out = _sc_pool(x.T.astype(jnp.int32), tp)
    return out.reshape(_C, _B).T


# speedup vs baseline: 1.0088x; 1.0032x over previous
import functools

import jax
import jax.numpy as jnp
from jax import lax
from jax.experimental import pallas as pl
from jax.experimental.pallas import tpu as pltpu
from jax.experimental.pallas import tpu_sc as plsc

_VOCAB = 1000000
_D = 64
_C = 2
_B = 4096
_SEQ = 200

_NC = 2
_NS = 16
_NW = _NC * _NS
_BPW = _B // _NW
_LANES = 16
_GPW = _BPW // _LANES

_NB = 32768
_GRID = (_VOCAB + _NB - 1) // _NB


def _proj_body(w_ref, b_ref, tt_ref, o_ref):
    scale = jnp.float32(1.0 / _SEQ)
    res = (jnp.dot(w_ref[...], tt_ref[...],
                   preferred_element_type=jnp.float32)
           + b_ref[...]) * scale
    au = lax.bitcast_convert_type(
        res[0, :].astype(jnp.bfloat16), jnp.uint16).astype(jnp.uint32)
    bu = lax.bitcast_convert_type(
        res[1, :].astype(jnp.bfloat16), jnp.uint16).astype(jnp.uint32)
    o_ref[...] = lax.bitcast_convert_type((au << 16) | bu, jnp.float32)


_proj = pl.pallas_call(
    _proj_body,
    grid=(_GRID,),
    in_specs=[
        pl.BlockSpec((_C, _D), lambda i: (0, 0)),
        pl.BlockSpec((_C, 1), lambda i: (0, 0)),
        pl.BlockSpec((_D, _NB), lambda i: (0, i)),
    ],
    out_specs=pl.BlockSpec((_NB,), lambda i: (i,)),
    out_shape=jax.ShapeDtypeStruct((_VOCAB,), jnp.float32),
)

_WAVE = 20
_NWAVE = _SEQ // _WAVE
_HIMASK = jnp.uint32(0xFFFF0000)


def _pool_body(xt_hbm, tp_hbm, out_hbm,
               idx_v, g_v, o0_v, o1_v, sem0, sem1):
    wid = lax.axis_index("s") * _NC + lax.axis_index("c")
    base = wid * _BPW

    pltpu.sync_copy(xt_hbm.at[:, pl.ds(base, _BPW)], idx_v)

    sems = (sem0, sem1)

    def copy(j, sem):
        return pltpu.make_async_copy(tp_hbm.at[idx_v.at[j]],
                                     g_v.at[pl.ds(j * _BPW, _BPW)], sem)

    def fire_wave(w, par):
        def f1(j, _):
            copy(j, sems[par]).start()
            return 0
        lax.fori_loop(w * _WAVE, (w + 1) * _WAVE, f1, 0)

    def wait_wave(w, par):
        def f1(j, _):
            copy(j, sems[par]).wait()
            return 0
        lax.fori_loop(w * _WAVE, (w + 1) * _WAVE, f1, 0)

    def reduce_wave(w, accs):
        def rbody(j, carry):
            acc = list(carry)
            for g in range(_GPW):
                u = plsc.bitcast(g_v[pl.ds(j * _BPW + g * _LANES, _LANES)],
                                 jnp.uint32)
                c0 = plsc.bitcast(u & _HIMASK, jnp.float32)
                c1 = plsc.bitcast(u << 16, jnp.float32)
                acc[g] = acc[g] + c0
                acc[_GPW + g] = acc[_GPW + g] + c1
            return tuple(acc)
        return lax.fori_loop(w * _WAVE, (w + 1) * _WAVE, rbody, accs)

    fire_wave(0, 0)
    zero = jnp.zeros((_LANES,), jnp.float32)

    def outer(i, carry):
        for par in range(2):
            w = 2 * i + par

            @pl.when(w + 1 < _NWAVE)
            def _fire():
                fire_wave(w + 1, 1 - par)

            wait_wave(w, par)
            carry = reduce_wave(w, carry)
        return carry

    acc = lax.fori_loop(0, _NWAVE // 2, outer, (zero,) * (2 * _GPW))
    for g in range(_GPW):
        o0_v[pl.ds(g * _LANES, _LANES)] = acc[g]
        o1_v[pl.ds(g * _LANES, _LANES)] = acc[_GPW + g]

    pltpu.sync_copy(o0_v, out_hbm.at[pl.ds(base, _BPW)])
    pltpu.sync_copy(o1_v, out_hbm.at[pl.ds(_B + base, _BPW)])


@functools.partial(
    pl.kernel,
    out_type=jax.ShapeDtypeStruct((_C * _B,), jnp.float32),
    mesh=plsc.VectorSubcoreMesh(core_axis_name="c", subcore_axis_name="s"),
    scratch_types=[
        pltpu.VMEM((_SEQ, _BPW), jnp.int32),
        pltpu.VMEM((_SEQ * _BPW,), jnp.float32),
        pltpu.VMEM((_BPW,), jnp.float32),
        pltpu.VMEM((_BPW,), jnp.float32),
        pltpu.SemaphoreType.DMA,
        pltpu.SemaphoreType.DMA,
    ],
    compiler_params=pltpu.CompilerParams(use_tc_tiling_on_sc=False,
                                         needs_layout_passes=False),
)
def _sc_pool(xt_hbm, tp_hbm, out_hbm,
             idx_v, g_v, o0_v, o1_v, sem0, sem1):
    _pool_body(xt_hbm, tp_hbm, out_hbm,
               idx_v, g_v, o0_v, o1_v, sem0, sem1)


def kernel(x, table, W, b):
    tp = _proj(W.astype(jnp.float32), b.astype(jnp.float32)[:, None],
               table.T)
    out = _sc_pool(x.T.astype(jnp.int32), tp)
    return out.reshape(_C, _B).T
